# Initial kernel scaffold; baseline (speedup 1.0000x reference)
import jax
import jax.numpy as jnp
from jax.experimental import pallas as pl


def kernel(feat, boxes, grid_embedding):
    raise NotImplementedError("write your pallas kernel here")


# speedup vs baseline: 3.7681x; 3.7681x over previous
import functools

import jax
import jax.numpy as jnp
from jax import lax
from jax.experimental import pallas as pl
from jax.experimental.pallas import tpu as pltpu
from jax.experimental.pallas import tpu_sc as plsc

PH, PW = 4, 4
SR = 2
NBINS = PH * PW

_F32 = jnp.float32
_I32 = jnp.int32


def _axis_corners_weights(coords, size):
  lane = lax.iota(_I32, 16)
  is_hi = lane >= 8
  valid = jnp.where(coords >= -1.0, 1.0, 0.0) * jnp.where(
      coords <= float(size), 1.0, 0.0)
  c = jnp.maximum(coords, 0.0)
  c0 = c.astype(_I32)
  c0f = c0.astype(_F32)
  hi = jnp.minimum(c0 + 1, size - 1)
  cond = c0 >= size - 1
  lo = jnp.where(cond, size - 1, c0)
  l = jnp.where(cond, 0.0, c - c0f)
  w_lo = (1.0 - l) * valid
  w_hi = l * valid
  corner = jnp.where(is_hi, hi, lo)
  weight = jnp.where(is_hi, w_hi, w_lo)
  return corner, weight


def _roi_body(featT, rois, ge_t, out, rois_v, ge_v, idx_v, rows_v, out_v,
              sem0, sem1, H, W, C, rois_per_w):
  nc = 2
  wid = lax.axis_index("s") * nc + lax.axis_index("c")
  base = wid * rois_per_w
  pltpu.sync_copy(rois.at[wid], rois_v)
  pltpu.sync_copy(ge_t, ge_v)

  lane = lax.iota(_I32, 16)
  k8 = lane & 7
  offs = (k8.astype(_F32) + 0.5) * 0.5
  nchunks = C // 16

  def one_roi(t, _):
    rv = rois_v[t, :]
    cx = rv[0]
    cy = rv[1]
    bh = rv[2]
    bw = rv[3]
    bbase = rv[4].astype(_I32)
    x1 = jnp.clip(cx - 0.5 * bw, 0.0, 1.0) * W
    x2 = jnp.clip(cx + 0.5 * bw, 0.0, 1.0) * W
    y1 = jnp.clip(cy - 0.5 * bh, 0.0, 1.0) * H
    y2 = jnp.clip(cy + 0.5 * bh, 0.0, 1.0) * H
    roi_w = jnp.maximum(x2 - x1, 1.0)
    roi_h = jnp.maximum(y2 - y1, 1.0)
    bin_w = roi_w * (1.0 / PW)
    bin_h = roi_h * (1.0 / PH)
    xs = x1 + offs * bin_w
    ys = y1 + offs * bin_h
    xc, xw = _axis_corners_weights(xs, W)
    yc, yw = _axis_corners_weights(ys, H)
    xw = xw * 0.25
    yterms = bbase + yc * W

    for i in range(16):
      idx_v[i // 8, pl.ds((i % 8) * 16, 16)] = yterms[i] + xc

    cp0 = pltpu.make_async_copy(featT.at[idx_v.at[0]],
                                rows_v.at[pl.ds(0, 128)], sem0)
    cp1 = pltpu.make_async_copy(featT.at[idx_v.at[1]],
                                rows_v.at[pl.ds(128, 128)], sem1)
    cp0.start()
    cp1.start()
    cp0.wait()
    cp1.wait()

    xws = [jnp.full((16,), xw[j], _F32) for j in range(16)]

    def chunk(kk, _):
      c0 = pl.multiple_of(kk * 16, 16)
      acc = [jnp.zeros((16,), _F32) for _ in range(NBINS)]
      for i in range(16):
        tpw = [jnp.zeros((16,), _F32) for _ in range(PW)]
        for j in range(16):
          row = rows_v[i * 16 + j, pl.ds(c0, 16)]
          tpw[(j % 8) // 2] = tpw[(j % 8) // 2] + xws[j] * row
        ywi = jnp.full((16,), yw[i], _F32)
        ph = (i % 8) // 2
        for pw in range(PW):
          b = ph * PW + pw
          acc[b] = acc[b] + ywi * tpw[pw]
      for b in range(NBINS):
        vals = acc[b] + ge_v[b, pl.ds(c0, 16)]
        out_v[b, pl.ds(c0, 16)] = vals
      return 0

    lax.fori_loop(0, nchunks, chunk, 0, unroll=False)
    pltpu.sync_copy(out_v, out.at[base + t])
    return 0

  lax.fori_loop(0, rois_per_w, one_roi, 0, unroll=False)


@jax.jit
def _roisample_sc(featT, rois, ge_t):
  nw, rois_per_w = rois.shape[0], rois.shape[1]
  R = nw * rois_per_w
  C = featT.shape[1]
  mesh = plsc.VectorSubcoreMesh(core_axis_name="c", subcore_axis_name="s")
  body = functools.partial(
      _roi_body, H=32, W=32, C=C, rois_per_w=rois_per_w)
  return pl.kernel(
      body,
      out_type=jax.ShapeDtypeStruct((R, NBINS, C), _F32),
      mesh=mesh,
      scratch_types=[
          pltpu.VMEM((rois_per_w, 16), _F32),
          pltpu.VMEM((NBINS, C), _F32),
          pltpu.VMEM((2, 128), _I32),
          pltpu.VMEM((256, C), _F32),
          pltpu.VMEM((NBINS, C), _F32),
          pltpu.SemaphoreType.DMA,
          pltpu.SemaphoreType.DMA,
      ],
  )(featT, rois, ge_t)


def kernel(feat, boxes, grid_embedding):
  B, C, H, W = feat.shape
  nq = boxes.shape[1]
  R = B * nq
  featT = jnp.transpose(feat, (0, 2, 3, 1)).reshape(B * H * W, C)
  bbase = (jnp.arange(R, dtype=_I32) // nq * (H * W)).astype(_F32)
  rois = jnp.concatenate(
      [boxes.reshape(R, 4), bbase[:, None],
       jnp.zeros((R, 11), _F32)], axis=1).reshape(32, R // 32, 16)
  ge_t = jnp.transpose(grid_embedding.reshape(C, NBINS), (1, 0))
  out = _roisample_sc(featT, rois, ge_t)
  out = jnp.transpose(out.reshape(B, nq, NBINS, C), (0, 1, 3, 2))
  return out.reshape(B, nq, C, PH, PW)
